# Initial kernel scaffold; baseline (speedup 1.0000x reference)
import jax
import jax.numpy as jnp
from jax.experimental import pallas as pl


def kernel(x, table):
    raise NotImplementedError("write your pallas kernel here")


# speedup vs baseline: 1.7976x; 1.7976x over previous
import functools

import jax
import jax.numpy as jnp
from jax import lax
from jax.experimental import pallas as pl
from jax.experimental.pallas import tpu as pltpu
from jax.experimental.pallas import tpu_sc as plsc

_VOCAB = 1000000
_EMBED = 64
_S = 128
_R = 4


@functools.cache
def _make_lookup(num_rows: int):
  info = plsc.get_sparse_core_info()
  nw = info.num_cores * info.num_subcores
  rows_per_w = num_rows // nw
  chunks = rows_per_w // _R
  assert rows_per_w % _R == 0

  mesh = plsc.VectorSubcoreMesh(core_axis_name="c", subcore_axis_name="s")

  @functools.partial(
      pl.kernel,
      out_type=jax.ShapeDtypeStruct((num_rows * _S, _EMBED), jnp.float32),
      mesh=mesh,
      scratch_types=[
          pltpu.VMEM((_R, _S), jnp.int32),
          pltpu.VMEM((_R * _S, _EMBED), jnp.float32),
          pltpu.SemaphoreType.DMA,
      ],
      compiler_params=pltpu.CompilerParams(use_tc_tiling_on_sc=False),
  )
  def lookup(idx_hbm, table_hbm, out_hbm, idx_v, rows_v, sem):
    wid = lax.axis_index("s") * info.num_cores + lax.axis_index("c")
    row0 = wid * rows_per_w

    def body(g, carry):
      r = row0 + g * _R
      pltpu.sync_copy(idx_hbm.at[pl.ds(r, _R)], idx_v)
      cps = [
          pltpu.async_copy(
              table_hbm.at[idx_v.at[j]],
              rows_v.at[pl.ds(j * _S, _S)],
              sem,
          )
          for j in range(_R)
      ]
      for c in cps:
        c.wait()
      pltpu.sync_copy(rows_v, out_hbm.at[pl.ds(r * _S, _R * _S)])
      return carry

    lax.fori_loop(0, chunks, body, 0)

  return lookup


def kernel(x, table):
  b, s = x.shape
  idx = x.astype(jnp.int32).reshape(b * s // _S, _S)
  out = _make_lookup(idx.shape[0])(idx, table)
  return out.reshape(b, s, _EMBED)


# speedup vs baseline: 1.8691x; 1.0398x over previous
import functools

import jax
import jax.numpy as jnp
from jax import lax
from jax.experimental import pallas as pl
from jax.experimental.pallas import tpu as pltpu
from jax.experimental.pallas import tpu_sc as plsc

_VOCAB = 1000000
_EMBED = 64
_S = 128
_R = 4


@functools.cache
def _make_lookup(num_rows: int):
  info = plsc.get_sparse_core_info()
  nw = info.num_cores * info.num_subcores
  rows_per_w = num_rows // nw
  chunks = rows_per_w // _R
  assert rows_per_w % _R == 0 and chunks % 2 == 0
  c_rows = _R * _S

  mesh = plsc.VectorSubcoreMesh(core_axis_name="c", subcore_axis_name="s")

  @functools.partial(
      pl.kernel,
      out_type=jax.ShapeDtypeStruct((num_rows * _S, _EMBED), jnp.float32),
      mesh=mesh,
      scratch_types=[
          pltpu.VMEM((_R, _S), jnp.int32),
          pltpu.VMEM((_R, _S), jnp.int32),
          pltpu.VMEM((c_rows, _EMBED), jnp.float32),
          pltpu.VMEM((c_rows, _EMBED), jnp.float32),
          pltpu.SemaphoreType.DMA,
          pltpu.SemaphoreType.DMA,
          pltpu.SemaphoreType.DMA,
          pltpu.SemaphoreType.DMA,
      ],
      compiler_params=pltpu.CompilerParams(use_tc_tiling_on_sc=False),
  )
  def lookup(idx_hbm, table_hbm, out_hbm, idx0, idx1, rows0, rows1,
             sg0, sg1, so0, so1):
    wid = lax.axis_index("s") * info.num_cores + lax.axis_index("c")
    row0 = wid * rows_per_w

    def fire_gathers(idx_v, rows_v, sem):
      for j in range(_R):
        pltpu.async_copy(
            table_hbm.at[idx_v.at[j]], rows_v.at[pl.ds(j * _S, _S)], sem)

    def drain_gathers(rows_v, sem):
      pltpu.make_async_copy(out_hbm.at[pl.ds(0, c_rows)], rows_v, sem).wait()

    def start_out(g, rows_v, sem):
      pltpu.async_copy(
          rows_v, out_hbm.at[pl.ds((row0 + g * _R) * _S, c_rows)], sem)

    def drain_out(rows_v, sem):
      pltpu.make_async_copy(
          rows_v, out_hbm.at[pl.ds(0, c_rows)], sem).wait()

    def load_idx(g, idx_v):
      pltpu.sync_copy(idx_hbm.at[pl.ds(row0 + g * _R, _R)], idx_v)

    load_idx(0, idx0)
    fire_gathers(idx0, rows0, sg0)
    load_idx(1, idx1)
    fire_gathers(idx1, rows1, sg1)

    def body(g2, carry):
      ga = 2 * g2
      drain_gathers(rows0, sg0)
      start_out(ga, rows0, so0)
      drain_gathers(rows1, sg1)
      start_out(ga + 1, rows1, so1)
      drain_out(rows0, so0)
      load_idx(ga + 2, idx0)
      fire_gathers(idx0, rows0, sg0)
      drain_out(rows1, so1)
      load_idx(ga + 3, idx1)
      fire_gathers(idx1, rows1, sg1)
      return carry

    lax.fori_loop(0, chunks // 2 - 1, body, 0)

    ga = chunks - 2
    drain_gathers(rows0, sg0)
    start_out(ga, rows0, so0)
    drain_gathers(rows1, sg1)
    start_out(ga + 1, rows1, so1)
    drain_out(rows0, so0)
    drain_out(rows1, so1)

  return lookup


def kernel(x, table):
  b, s = x.shape
  idx = x.astype(jnp.int32).reshape(b * s // _S, _S)
  out = _make_lookup(idx.shape[0])(idx, table)
  return out.reshape(b, s, _EMBED)
